# baseline (speedup 1.0000x reference)
import functools

import jax
import jax.numpy as jnp
from jax import lax
from jax.experimental import pallas as pl
from jax.experimental.pallas import tpu as pltpu
from jax.experimental.pallas import tpu_sc as plsc

_NC = 2
_NS = 16
_NW = _NC * _NS


def _conv9_body(x_ref, w_ref, b_ref, o_ref, *, H, Wp, relu, precision):
    M = H * Wp
    cout = o_ref.shape[-1]
    acc = jnp.zeros((M, cout), jnp.float32)
    for oy in range(3):
        for ox in range(3):
            start = oy * Wp + ox
            xs = x_ref[0, pl.ds(start, M), :]
            acc = acc + jnp.dot(xs, w_ref[3 * oy + ox],
                                precision=precision,
                                preferred_element_type=jnp.float32)
    acc = acc + b_ref[0][None, :]
    if relu:
        acc = jnp.maximum(acc, 0.0)
    o_ref[0] = acc


def _conv9(xpf, wt, b, H, relu, precision=lax.Precision.DEFAULT):
    N, Mp, cin = xpf.shape
    cout = wt.shape[-1]
    W = H
    Wp = W + 2
    out = pl.pallas_call(
        functools.partial(_conv9_body, H=H, Wp=Wp, relu=relu,
                          precision=precision),
        grid=(N,),
        in_specs=[
            pl.BlockSpec((1, Mp, cin), lambda n: (n, 0, 0)),
            pl.BlockSpec((9, cin, cout), lambda n: (0, 0, 0)),
            pl.BlockSpec((1, cout), lambda n: (0, 0)),
        ],
        out_specs=pl.BlockSpec((1, H * Wp, cout), lambda n: (n, 0, 0)),
        out_shape=jax.ShapeDtypeStruct((N, H * Wp, cout), jnp.float32),
    )(xpf, wt, b.reshape(1, cout))
    return out.reshape(N, H, Wp, cout)[:, :, :W, :]


def _pad_flat(x):
    N, H, W, C = x.shape
    xp = jnp.pad(x, ((0, 0), (1, 1), (1, 1), (0, 0)))
    rows = (H + 2) * (W + 2)
    tot = ((rows + 2 + 7) // 8) * 8
    xf = xp.reshape(N, rows, C)
    return jnp.pad(xf, ((0, 0), (0, tot - rows), (0, 0)))


def _s2d(x):
    N, H, W, C = x.shape
    x = x.reshape(N, H // 2, 2, W // 2, 2, C)
    x = jnp.transpose(x, (0, 1, 3, 2, 4, 5))
    return x.reshape(N, H // 2, W // 2, 4 * C)


def _d2s(y, cout):
    N, H, W, _ = y.shape
    y = y.reshape(N, H, W, 2, 2, cout)
    y = jnp.transpose(y, (0, 1, 3, 2, 4, 5))
    return y.reshape(N, 2 * H, 2 * W, cout)


def _fold_s2_w(w):
    cout, cin = w.shape[0], w.shape[1]
    wt = jnp.zeros((3, 3, 2, 2, cin, cout), jnp.float32)
    for oy in range(3):
        for ox in range(3):
            for py in range(2):
                for px in range(2):
                    ty = 2 * oy - 1 + py
                    tx = 2 * ox - 1 + px
                    if 0 <= ty < 4 and 0 <= tx < 4:
                        wt = wt.at[oy, ox, py, px].set(
                            jnp.transpose(w[:, :, ty, tx]))
    return wt.reshape(9, 4 * cin, cout)


def _fold_t2_w(dw):
    w_ = jnp.transpose(dw[:, :, ::-1, ::-1], (1, 0, 2, 3))
    cout, cin = w_.shape[0], w_.shape[1]
    wt = jnp.zeros((3, 3, cin, 2, 2, cout), jnp.float32)
    for oy in range(3):
        for ox in range(3):
            for r in range(2):
                for s in range(2):
                    u, v = oy - r, ox - s
                    if 0 <= u < 2 and 0 <= v < 2:
                        wt = wt.at[oy, ox, :, r, s, :].set(
                            jnp.transpose(w_[:, :, r + 2 * u, s + 2 * v]))
    return wt.reshape(9, cin, 4 * cout)


def _vq_body(z_ref, et_ref, idx_ref):
    et = et_ref[...]
    e_sq = jnp.sum(et * et, axis=0)
    z = z_ref[...]
    s = jnp.dot(z, et, preferred_element_type=jnp.float32)
    z_sq = jnp.sum(z * z, axis=1, keepdims=True)
    d = e_sq[None, :] + z_sq - 2.0 * s
    m = jnp.min(d, axis=1, keepdims=True)
    iota = lax.broadcasted_iota(jnp.int32, d.shape, 1)
    idx = jnp.min(jnp.where(d <= m, iota, jnp.int32(d.shape[1])), axis=1)
    idx_ref[0, 0, :] = idx


def _vq_argmin(ze_flat, emb):
    M, C = ze_flat.shape
    K = emb.shape[0]
    Mt = 256
    G = M // Mt
    emb_t = jnp.transpose(emb)
    out = pl.pallas_call(
        _vq_body,
        grid=(G,),
        in_specs=[
            pl.BlockSpec((Mt, C), lambda g: (g, 0)),
            pl.BlockSpec((C, K), lambda g: (0, 0)),
        ],
        out_specs=pl.BlockSpec((1, 1, Mt), lambda g: (g, 0, 0)),
        out_shape=jax.ShapeDtypeStruct((G, 1, Mt), jnp.int32),
    )(ze_flat, emb_t)
    return out.reshape(M)


def _vq_gather(emb, idx):
    D0 = emb.shape[1]
    emb = jnp.pad(emb, ((0, 0), (0, 128 - D0)))
    B, D = idx.shape[0], 128
    bpw = B // _NW
    mesh = plsc.VectorSubcoreMesh(core_axis_name="c", subcore_axis_name="s")

    @functools.partial(
        pl.kernel,
        mesh=mesh,
        out_type=jax.ShapeDtypeStruct((B, D), jnp.float32),
        scratch_types=[
            pltpu.VMEM((bpw,), jnp.int32),
            pltpu.VMEM((bpw, D), jnp.float32),
            pltpu.SemaphoreType.DMA,
        ],
    )
    def k(table_hbm, idx_hbm, out_hbm, idx_v, rows_v, sem):
        wid = lax.axis_index("s") * _NC + lax.axis_index("c")
        base = wid * bpw
        pltpu.sync_copy(idx_hbm.at[pl.ds(base, bpw)], idx_v)
        pltpu.async_copy(table_hbm.at[idx_v], rows_v, sem).wait()
        pltpu.sync_copy(rows_v, out_hbm.at[pl.ds(base, bpw)])

    return k(emb, idx)[:, :D0]


def kernel(x, w1, b1, w2, b2, w3, b3, emb, dw1, db1, dw2, db2, dw3, db3):
    N = x.shape[0]

    w1t = _fold_s2_w(w1)
    w2t = _fold_s2_w(w2)
    w3t = jnp.transpose(w3, (2, 3, 1, 0)).reshape(9, 64, 64)
    u1 = jnp.transpose(dw1[:, :, ::-1, ::-1], (1, 0, 2, 3))
    w4t = jnp.transpose(u1, (2, 3, 1, 0)).reshape(9, 64, 32)
    w5t = _fold_t2_w(dw2)
    w6t = _fold_t2_w(dw3)

    xh = jnp.transpose(x, (0, 2, 3, 1))
    s0 = _s2d(xh)
    h1 = _conv9(_pad_flat(s0), w1t, b1, 112, relu=True)
    s1 = _s2d(h1)
    h2 = _conv9(_pad_flat(s1), w2t, b2, 56, relu=True)
    ze = _conv9(_pad_flat(h2), w3t, b3, 56, relu=True)

    ze_flat = ze.reshape(N * 56 * 56, 64)
    idx = _vq_argmin(ze_flat, emb)
    zq_flat = _vq_gather(emb, idx)
    di = ze_flat + (zq_flat - ze_flat)
    di = di.reshape(N, 56, 56, 64)

    d1 = _conv9(_pad_flat(di), w4t, db1, 56, relu=True)
    d2p = _conv9(_pad_flat(d1), w5t, jnp.tile(db2, 4), 56,
                 relu=True)
    d2 = _d2s(d2p, 64)
    y3 = _conv9(_pad_flat(d2), w6t, jnp.tile(db3, 4), 112,
                relu=False)
    x_hat = jnp.transpose(_d2s(y3, 3), (0, 3, 1, 2))

    zq = jnp.transpose(zq_flat.reshape(N, 56, 56, 64), (0, 3, 1, 2))
    ze_out = jnp.transpose(ze, (0, 3, 1, 2))
    return (x_hat, ze_out, zq)
